# Initial kernel scaffold; baseline (speedup 1.0000x reference)
import jax
import jax.numpy as jnp
from jax.experimental import pallas as pl


def kernel(x, idx, target_mean_weights):
    raise NotImplementedError("write your pallas kernel here")


# speedup vs baseline: 122.2333x; 122.2333x over previous
import functools

import jax
import jax.numpy as jnp
from jax import lax
from jax.experimental import pallas as pl
from jax.experimental.pallas import tpu as pltpu
from jax.experimental.pallas import tpu_sc as plsc

E = 6_400_000
NSEG = 200
STRENGTH = 0.001

NC = 2
NS = 16
NW = NC * NS
LANES = 16

ACC = 256
PER_TILE = E // NW
CHUNK = 20_000
NCHUNK = PER_TILE // CHUNK
VECS = CHUNK // LANES


def _sc_partials(x, idx):
    mesh = plsc.VectorSubcoreMesh(core_axis_name="c", subcore_axis_name="s")

    @functools.partial(
        pl.kernel,
        mesh=mesh,
        compiler_params=pltpu.CompilerParams(needs_layout_passes=False),
        out_type=jax.ShapeDtypeStruct((2, NW, ACC), jnp.float32),
        scratch_types=[
            pltpu.VMEM((CHUNK,), jnp.float32),
            pltpu.VMEM((CHUNK,), jnp.float32),
            pltpu.VMEM((CHUNK,), jnp.int32),
            pltpu.VMEM((CHUNK,), jnp.int32),
            pltpu.VMEM((ACC,), jnp.float32),
            pltpu.VMEM((ACC,), jnp.float32),
            pltpu.SemaphoreType.DMA,
            pltpu.SemaphoreType.DMA,
        ],
    )
    def k(x_hbm, idx_hbm, out_hbm, x_buf0, x_buf1, i_buf0, i_buf1,
          acc_s, acc_c, sem0, sem1):
        wid = lax.axis_index("s") * NC + lax.axis_index("c")
        base = wid * PER_TILE
        sems = (sem0, sem1)
        x_bufs = (x_buf0, x_buf1)
        i_bufs = (i_buf0, i_buf1)

        zeros16 = jnp.zeros((LANES,), jnp.float32)
        for j in range(ACC // LANES):
            acc_s[pl.ds(j * LANES, LANES)] = zeros16
            acc_c[pl.ds(j * LANES, LANES)] = zeros16

        def start(c):
            slot = c % 2
            off = base + c * CHUNK
            hx = pltpu.async_copy(
                x_hbm.at[pl.ds(off, CHUNK)], x_bufs[slot], sems[slot])
            hi = pltpu.async_copy(
                idx_hbm.at[pl.ds(off, CHUNK)], i_bufs[slot], sems[slot])
            return (hx, hi)

        ones16 = jnp.ones((LANES,), jnp.float32)
        handles = [None] * NCHUNK
        handles[0] = start(0)
        for c in range(NCHUNK):
            if c + 1 < NCHUNK:
                handles[c + 1] = start(c + 1)
            for h in handles[c]:
                h.wait()
            slot = c % 2

            def body(i, carry):
                xv = x_bufs[slot][pl.ds(i * LANES, LANES)]
                iv = i_bufs[slot][pl.ds(i * LANES, LANES)]
                plsc.addupdate_scatter(acc_s, [iv], xv)
                plsc.addupdate_scatter(acc_c, [iv], ones16)
                return carry

            lax.fori_loop(0, VECS, body, 0)

        pltpu.sync_copy(acc_s, out_hbm.at[0, wid])
        pltpu.sync_copy(acc_c, out_hbm.at[1, wid])

    return k(x, idx)


def _loss_tc(partials, target_pad):
    def body(p_ref, t_ref, o_ref):
        p = p_ref[...]
        sums = jnp.sum(p[0], axis=0, keepdims=True)
        cnts = jnp.sum(p[1], axis=0, keepdims=True)
        mean = sums / jnp.maximum(cnts, 1.0)
        d = mean - t_ref[...]
        col = lax.broadcasted_iota(jnp.int32, (1, ACC), 1)
        sq = jnp.where(col < NSEG, d * d, 0.0)
        loss = jnp.sum(sq) * jnp.float32(STRENGTH / NSEG)
        o_ref[...] = jnp.broadcast_to(loss, (1, 1))

    return pl.pallas_call(
        body,
        out_shape=jax.ShapeDtypeStruct((1, 1), jnp.float32),
    )(partials, target_pad)


def kernel(x, idx, target_mean_weights):
    partials = _sc_partials(x, idx)
    tgt = jnp.pad(target_mean_weights, (0, ACC - NSEG)).reshape(1, ACC)
    loss = _loss_tc(partials, tgt)
    return loss[0, 0].astype(jnp.float32)


# speedup vs baseline: 192.5751x; 1.5755x over previous
import functools

import jax
import jax.numpy as jnp
from jax import lax
from jax.experimental import pallas as pl
from jax.experimental.pallas import tpu as pltpu
from jax.experimental.pallas import tpu_sc as plsc

E = 6_400_000
NSEG = 200
STRENGTH = 0.001

NC = 2
NS = 16
NW = NC * NS
LANES = 16

ACC = 256
PER_TILE = E // NW
CHUNK = 20_000
NCHUNK = PER_TILE // CHUNK
VECS = CHUNK // LANES


def _sc_partials(x, idx):
    mesh = plsc.VectorSubcoreMesh(core_axis_name="c", subcore_axis_name="s")

    @functools.partial(
        pl.kernel,
        mesh=mesh,
        compiler_params=pltpu.CompilerParams(needs_layout_passes=False),
        out_type=jax.ShapeDtypeStruct((2, NW, ACC), jnp.float32),
        scratch_types=[
            pltpu.VMEM((CHUNK,), jnp.float32),
            pltpu.VMEM((CHUNK,), jnp.float32),
            pltpu.VMEM((CHUNK,), jnp.int32),
            pltpu.VMEM((CHUNK,), jnp.int32),
            pltpu.VMEM((ACC,), jnp.float32),
            pltpu.VMEM((ACC,), jnp.float32),
            pltpu.SemaphoreType.DMA,
            pltpu.SemaphoreType.DMA,
        ],
    )
    def k(x_hbm, idx_hbm, out_hbm, x_buf0, x_buf1, i_buf0, i_buf1,
          acc_s, acc_c, sem0, sem1):
        wid = lax.axis_index("s") * NC + lax.axis_index("c")
        base = wid * PER_TILE
        sems = (sem0, sem1)
        x_bufs = (x_buf0, x_buf1)
        i_bufs = (i_buf0, i_buf1)

        zeros16 = jnp.zeros((LANES,), jnp.float32)
        for j in range(ACC // LANES):
            acc_s[pl.ds(j * LANES, LANES)] = zeros16
            acc_c[pl.ds(j * LANES, LANES)] = zeros16

        def start(c):
            slot = c % 2
            off = base + c * CHUNK
            hx = pltpu.async_copy(
                x_hbm.at[pl.ds(off, CHUNK)], x_bufs[slot], sems[slot])
            hi = pltpu.async_copy(
                idx_hbm.at[pl.ds(off, CHUNK)], i_bufs[slot], sems[slot])
            return (hx, hi)

        ones16 = jnp.ones((LANES,), jnp.float32)
        handles = [None] * NCHUNK
        handles[0] = start(0)
        for c in range(NCHUNK):
            if c + 1 < NCHUNK:
                handles[c + 1] = start(c + 1)
            for h in handles[c]:
                h.wait()
            slot = c % 2

            @plsc.parallel_loop(0, CHUNK, step=LANES, unroll=8)
            def _(i):
                xv = x_bufs[slot][pl.ds(i, LANES)]
                iv = i_bufs[slot][pl.ds(i, LANES)]
                plsc.addupdate_scatter(acc_s, [iv], xv)
                plsc.addupdate_scatter(acc_c, [iv], ones16)

        pltpu.sync_copy(acc_s, out_hbm.at[0, wid])
        pltpu.sync_copy(acc_c, out_hbm.at[1, wid])

    return k(x, idx)


def _loss_tc(partials, target_pad):
    def body(p_ref, t_ref, o_ref):
        p = p_ref[...]
        sums = jnp.sum(p[0], axis=0, keepdims=True)
        cnts = jnp.sum(p[1], axis=0, keepdims=True)
        mean = sums / jnp.maximum(cnts, 1.0)
        d = mean - t_ref[...]
        col = lax.broadcasted_iota(jnp.int32, (1, ACC), 1)
        sq = jnp.where(col < NSEG, d * d, 0.0)
        loss = jnp.sum(sq) * jnp.float32(STRENGTH / NSEG)
        o_ref[...] = jnp.broadcast_to(loss, (1, 1))

    return pl.pallas_call(
        body,
        out_shape=jax.ShapeDtypeStruct((1, 1), jnp.float32),
    )(partials, target_pad)


def kernel(x, idx, target_mean_weights):
    partials = _sc_partials(x, idx)
    tgt = jnp.pad(target_mean_weights, (0, ACC - NSEG)).reshape(1, ACC)
    loss = _loss_tc(partials, tgt)
    return loss[0, 0].astype(jnp.float32)
